# baseline (speedup 1.0000x reference)
import jax
import jax.numpy as jnp
from jax import lax
from jax.experimental import pallas as pl
from jax.experimental.pallas import tpu as pltpu
from jax.experimental.pallas import tpu_sc as plsc

_BH = 1024


def _sc_gather_add(yf_hbm, idx_hbm, xb_hbm, out_hbm, idx_v, val_v, xb_v, sem):
    cid = lax.axis_index("c")
    sid = lax.axis_index("s")

    @pl.when((cid == 0) & (sid == 0))
    def _():
        pltpu.sync_copy(idx_hbm, idx_v)
        pltpu.sync_copy(xb_hbm, xb_v)
        pltpu.async_copy(yf_hbm.at[idx_v], val_v, sem).wait()
        val_v[...] = val_v[...] + xb_v[...]
        pltpu.sync_copy(val_v, out_hbm)


def _injected_values(Y, X, x, y):
    n, h, w = Y.shape
    idx = (jnp.arange(n, dtype=jnp.int32) * (h * w) + y * w + x).astype(jnp.int32)
    xb = jnp.broadcast_to(X, (n,))
    call = pl.kernel(
        _sc_gather_add,
        out_type=jax.ShapeDtypeStruct((n,), jnp.float32),
        mesh=plsc.VectorSubcoreMesh(core_axis_name="c", subcore_axis_name="s"),
        scratch_types=[
            pltpu.VMEM((n,), jnp.int32),
            pltpu.VMEM((n,), jnp.float32),
            pltpu.VMEM((n,), jnp.float32),
            pltpu.SemaphoreType.DMA,
        ],
    )
    return call(Y.reshape(-1), idx, xb)


def _copy_body(x_ref, y_ref, v_ref, y_blk, o_blk):
    i = pl.program_id(0)
    j = pl.program_id(1)
    o_blk[...] = y_blk[...]
    r_loc = y_ref[i] - j * _BH
    c = x_ref[i]

    @pl.when((r_loc >= 0) & (r_loc < _BH))
    def _inject():
        row = y_blk[0, pl.ds(r_loc, 1), :]
        w = row.shape[-1]
        colmask = jax.lax.broadcasted_iota(jnp.int32, (1, w), 1) == c
        o_blk[0, pl.ds(r_loc, 1), :] = jnp.where(colmask, v_ref[i], row)


def kernel(Y, X, x, y):
    n, h, w = Y.shape
    vals = _injected_values(Y, X, x, y)
    return pl.pallas_call(
        _copy_body,
        grid=(n, h // _BH),
        in_specs=[
            pl.BlockSpec(memory_space=pltpu.SMEM),
            pl.BlockSpec(memory_space=pltpu.SMEM),
            pl.BlockSpec(memory_space=pltpu.SMEM),
            pl.BlockSpec((1, _BH, w), lambda i, j: (i, j, 0)),
        ],
        out_specs=pl.BlockSpec((1, _BH, w), lambda i, j: (i, j, 0)),
        out_shape=jax.ShapeDtypeStruct(Y.shape, Y.dtype),
        compiler_params=pltpu.CompilerParams(
            dimension_semantics=("parallel", "parallel"),
        ),
    )(x, y, vals, Y)


# speedup vs baseline: 1.9773x; 1.9773x over previous
import jax
import jax.numpy as jnp
from jax import lax
from jax.experimental import pallas as pl
from jax.experimental.pallas import tpu as pltpu
from jax.experimental.pallas import tpu_sc as plsc

_BH = 1024


def _sc_gather_add(
    y2d_hbm, ridx_hbm, x_hbm, xb_hbm, out_hbm,
    ridx_v, x_v, rows_v, xb_v, val_v, sem,
):
    cid = lax.axis_index("c")
    sid = lax.axis_index("s")

    @pl.when((cid == 0) & (sid == 0))
    def _():
        n = ridx_v.shape[0]
        pltpu.sync_copy(ridx_hbm, ridx_v)
        pltpu.sync_copy(x_hbm, x_v)
        pltpu.sync_copy(xb_hbm, xb_v)
        pltpu.async_copy(y2d_hbm.at[ridx_v], rows_v, sem).wait()
        lanes = lax.iota(jnp.int32, n)
        vals = plsc.load_gather(rows_v, [lanes, x_v[...]])
        val_v[...] = vals + xb_v[...]
        pltpu.sync_copy(val_v, out_hbm)


def _injected_values(Y, X, x, y):
    n, h, w = Y.shape
    ridx = (jnp.arange(n, dtype=jnp.int32) * h + y).astype(jnp.int32)
    xb = jnp.broadcast_to(X, (n,))
    call = pl.kernel(
        _sc_gather_add,
        out_type=jax.ShapeDtypeStruct((n,), jnp.float32),
        mesh=plsc.VectorSubcoreMesh(core_axis_name="c", subcore_axis_name="s"),
        scratch_types=[
            pltpu.VMEM((n,), jnp.int32),
            pltpu.VMEM((n,), jnp.int32),
            pltpu.VMEM((n, w), jnp.float32),
            pltpu.VMEM((n,), jnp.float32),
            pltpu.VMEM((n,), jnp.float32),
            pltpu.SemaphoreType.DMA,
        ],
        compiler_params=pltpu.CompilerParams(needs_layout_passes=False),
    )
    return call(Y.reshape(n * h, w), ridx, x, xb)


def _copy_body(x_ref, y_ref, v_ref, y_blk, o_blk):
    i = pl.program_id(0)
    j = pl.program_id(1)
    o_blk[...] = y_blk[...]
    r_loc = y_ref[i] - j * _BH
    c = x_ref[i]

    @pl.when((r_loc >= 0) & (r_loc < _BH))
    def _inject():
        row = y_blk[0, pl.ds(r_loc, 1), :]
        w = row.shape[-1]
        colmask = jax.lax.broadcasted_iota(jnp.int32, (1, w), 1) == c
        o_blk[0, pl.ds(r_loc, 1), :] = jnp.where(colmask, v_ref[i], row)


def kernel(Y, X, x, y):
    n, h, w = Y.shape
    vals = _injected_values(Y, X, x, y)
    return pl.pallas_call(
        _copy_body,
        grid=(n, h // _BH),
        in_specs=[
            pl.BlockSpec(memory_space=pltpu.SMEM),
            pl.BlockSpec(memory_space=pltpu.SMEM),
            pl.BlockSpec(memory_space=pltpu.SMEM),
            pl.BlockSpec((1, _BH, w), lambda i, j: (i, j, 0)),
        ],
        out_specs=pl.BlockSpec((1, _BH, w), lambda i, j: (i, j, 0)),
        out_shape=jax.ShapeDtypeStruct(Y.shape, Y.dtype),
        compiler_params=pltpu.CompilerParams(
            dimension_semantics=("parallel", "parallel"),
        ),
    )(x, y, vals, Y)


# speedup vs baseline: 2.0068x; 1.0149x over previous
import jax
import jax.numpy as jnp
from jax import lax
from jax.experimental import pallas as pl
from jax.experimental.pallas import tpu as pltpu
from jax.experimental.pallas import tpu_sc as plsc

_BH = 1024


def _sc_gather_add(
    y2d_hbm, ridx_hbm, x_hbm, xb_hbm, out_hbm,
    ridx_v, x_v, rows_v, xb_v, val_v, sem,
):
    cid = lax.axis_index("c")
    sid = lax.axis_index("s")

    @pl.when((cid == 0) & (sid == 0))
    def _():
        n = ridx_v.shape[0]
        pltpu.sync_copy(ridx_hbm, ridx_v)
        pltpu.sync_copy(x_hbm, x_v)
        pltpu.sync_copy(xb_hbm, xb_v)
        pltpu.async_copy(y2d_hbm.at[ridx_v], rows_v, sem).wait()
        lanes = lax.iota(jnp.int32, n)
        vals = plsc.load_gather(rows_v, [lanes, x_v[...]])
        val_v[...] = vals + xb_v[...]
        pltpu.sync_copy(val_v, out_hbm)


def _injected_values(Y, X, x, y):
    n, h, w = Y.shape
    ridx = (jnp.arange(n, dtype=jnp.int32) * h + y).astype(jnp.int32)
    xb = jnp.broadcast_to(X, (n,))
    call = pl.kernel(
        _sc_gather_add,
        out_type=jax.ShapeDtypeStruct((n,), jnp.float32),
        mesh=plsc.VectorSubcoreMesh(core_axis_name="c", subcore_axis_name="s"),
        scratch_types=[
            pltpu.VMEM((n,), jnp.int32),
            pltpu.VMEM((n,), jnp.int32),
            pltpu.VMEM((n, w), jnp.float32),
            pltpu.VMEM((n,), jnp.float32),
            pltpu.VMEM((n,), jnp.float32),
            pltpu.SemaphoreType.DMA,
        ],
        compiler_params=pltpu.CompilerParams(needs_layout_passes=False),
    )
    return call(Y.reshape(n * h, w), ridx, x, xb)


def _copy_body(y_blk, o_blk):
    o_blk[...] = y_blk[...]


def _patch_body(x_ref, y_ref, v_ref, c_hbm, o_hbm, rowbuf, sem):
    n = rowbuf.shape[0]
    w = rowbuf.shape[-1]
    reads = []
    for i in range(n):
        d = pltpu.make_async_copy(
            o_hbm.at[i, pl.ds(y_ref[i], 1), :], rowbuf.at[i], sem
        )
        d.start()
        reads.append(d)
    cols = jax.lax.broadcasted_iota(jnp.int32, (1, w), 1)
    writes = []
    for i in range(n):
        reads[i].wait()
        rowbuf[i] = jnp.where(cols == x_ref[i], v_ref[i], rowbuf[i])
        d = pltpu.make_async_copy(
            rowbuf.at[i], o_hbm.at[i, pl.ds(y_ref[i], 1), :], sem
        )
        d.start()
        writes.append(d)
    for d in writes:
        d.wait()


def kernel(Y, X, x, y):
    n, h, w = Y.shape
    vals = _injected_values(Y, X, x, y)
    clone = pl.pallas_call(
        _copy_body,
        grid=(n, h // _BH),
        in_specs=[pl.BlockSpec((1, _BH, w), lambda i, j: (i, j, 0))],
        out_specs=pl.BlockSpec((1, _BH, w), lambda i, j: (i, j, 0)),
        out_shape=jax.ShapeDtypeStruct(Y.shape, Y.dtype),
        compiler_params=pltpu.CompilerParams(
            dimension_semantics=("parallel", "parallel"),
        ),
    )(Y)
    return pl.pallas_call(
        _patch_body,
        in_specs=[
            pl.BlockSpec(memory_space=pltpu.SMEM),
            pl.BlockSpec(memory_space=pltpu.SMEM),
            pl.BlockSpec(memory_space=pltpu.SMEM),
            pl.BlockSpec(memory_space=pl.ANY),
        ],
        out_specs=pl.BlockSpec(memory_space=pl.ANY),
        out_shape=jax.ShapeDtypeStruct(Y.shape, Y.dtype),
        input_output_aliases={3: 0},
        scratch_shapes=[
            pltpu.VMEM((n, 1, w), jnp.float32),
            pltpu.SemaphoreType.DMA,
        ],
    )(x, y, vals, clone)


# speedup vs baseline: 2.0071x; 1.0002x over previous
import jax
import jax.numpy as jnp
from jax import lax
from jax.experimental import pallas as pl
from jax.experimental.pallas import tpu as pltpu
from jax.experimental.pallas import tpu_sc as plsc

_BH = 1024


def _sc_gather_add(
    y2d_hbm, ridx_hbm, x_hbm, xb_hbm, out_hbm,
    ridx_v, x_v, rows_v, xb_v, val_v, sem,
):
    cid = lax.axis_index("c")
    sid = lax.axis_index("s")

    @pl.when((cid == 0) & (sid == 0))
    def _():
        n = ridx_v.shape[0]
        pltpu.sync_copy(ridx_hbm, ridx_v)
        pltpu.sync_copy(x_hbm, x_v)
        pltpu.sync_copy(xb_hbm, xb_v)
        pltpu.async_copy(y2d_hbm.at[ridx_v], rows_v, sem).wait()
        lanes = lax.iota(jnp.int32, n)
        vals = plsc.load_gather(rows_v, [lanes, x_v[...]])
        val_v[...] = vals + xb_v[...]
        pltpu.sync_copy(val_v, out_hbm)


def _injected_values(Y, X, x, y):
    n, h, w = Y.shape
    ridx = (jnp.arange(n, dtype=jnp.int32) * h + y).astype(jnp.int32)
    xb = jnp.broadcast_to(X, (n,))
    call = pl.kernel(
        _sc_gather_add,
        out_type=jax.ShapeDtypeStruct((n,), jnp.float32),
        mesh=plsc.VectorSubcoreMesh(core_axis_name="c", subcore_axis_name="s"),
        scratch_types=[
            pltpu.VMEM((n,), jnp.int32),
            pltpu.VMEM((n,), jnp.int32),
            pltpu.VMEM((n, w), jnp.float32),
            pltpu.VMEM((n,), jnp.float32),
            pltpu.VMEM((n,), jnp.float32),
            pltpu.SemaphoreType.DMA,
        ],
        compiler_params=pltpu.CompilerParams(
            needs_layout_passes=False,
            skip_device_barrier=True,
        ),
    )
    return call(Y.reshape(n * h, w), ridx, x, xb)


def _copy_body(y_blk, o_blk):
    o_blk[...] = y_blk[...]


def _patch_body(x_ref, y_ref, v_ref, c_hbm, o_hbm, rowbuf, sem):
    n = rowbuf.shape[0]
    w = rowbuf.shape[-1]
    reads = []
    for i in range(n):
        d = pltpu.make_async_copy(
            o_hbm.at[i, pl.ds(y_ref[i], 1), :], rowbuf.at[i], sem
        )
        d.start()
        reads.append(d)
    cols = jax.lax.broadcasted_iota(jnp.int32, (1, w), 1)
    writes = []
    for i in range(n):
        reads[i].wait()
        rowbuf[i] = jnp.where(cols == x_ref[i], v_ref[i], rowbuf[i])
        d = pltpu.make_async_copy(
            rowbuf.at[i], o_hbm.at[i, pl.ds(y_ref[i], 1), :], sem
        )
        d.start()
        writes.append(d)
    for d in writes:
        d.wait()


def kernel(Y, X, x, y):
    n, h, w = Y.shape
    vals = _injected_values(Y, X, x, y)
    clone = pl.pallas_call(
        _copy_body,
        grid=(n, h // _BH),
        in_specs=[pl.BlockSpec((1, _BH, w), lambda i, j: (i, j, 0))],
        out_specs=pl.BlockSpec((1, _BH, w), lambda i, j: (i, j, 0)),
        out_shape=jax.ShapeDtypeStruct(Y.shape, Y.dtype),
        compiler_params=pltpu.CompilerParams(
            dimension_semantics=("parallel", "parallel"),
        ),
    )(Y)
    return pl.pallas_call(
        _patch_body,
        in_specs=[
            pl.BlockSpec(memory_space=pltpu.SMEM),
            pl.BlockSpec(memory_space=pltpu.SMEM),
            pl.BlockSpec(memory_space=pltpu.SMEM),
            pl.BlockSpec(memory_space=pl.ANY),
        ],
        out_specs=pl.BlockSpec(memory_space=pl.ANY),
        out_shape=jax.ShapeDtypeStruct(Y.shape, Y.dtype),
        input_output_aliases={3: 0},
        scratch_shapes=[
            pltpu.VMEM((n, 1, w), jnp.float32),
            pltpu.SemaphoreType.DMA,
        ],
    )(x, y, vals, clone)


# speedup vs baseline: 2.2042x; 1.0982x over previous
import jax
import jax.numpy as jnp
from jax import lax
from jax.experimental import pallas as pl
from jax.experimental.pallas import tpu as pltpu
from jax.experimental.pallas import tpu_sc as plsc

_BH = 1024


def _sc_gather_add(
    y2d_hbm, ridx_hbm, x_hbm, xb_hbm, out_hbm,
    ridx_v, x_v, rows_v, xb_v, val_v, sem,
):
    cid = lax.axis_index("c")
    sid = lax.axis_index("s")

    @pl.when((cid == 0) & (sid == 0))
    def _():
        n = ridx_v.shape[0]
        pltpu.sync_copy(ridx_hbm, ridx_v)
        pltpu.sync_copy(x_hbm, x_v)
        pltpu.sync_copy(xb_hbm, xb_v)
        pltpu.async_copy(y2d_hbm.at[ridx_v], rows_v, sem).wait()
        lanes = lax.iota(jnp.int32, n)
        vals = plsc.load_gather(rows_v, [lanes, x_v[...]])
        val_v[...] = vals + xb_v[...]
        pltpu.sync_copy(val_v, out_hbm)


def _injected_values(Y, X, x, y):
    n, h, w = Y.shape
    ridx = (jnp.arange(n, dtype=jnp.int32) * h + y).astype(jnp.int32)
    xb = jnp.broadcast_to(X, (n,))
    call = pl.kernel(
        _sc_gather_add,
        out_type=jax.ShapeDtypeStruct((n,), jnp.float32),
        mesh=plsc.VectorSubcoreMesh(core_axis_name="c", subcore_axis_name="s"),
        scratch_types=[
            pltpu.VMEM((n,), jnp.int32),
            pltpu.VMEM((n,), jnp.int32),
            pltpu.VMEM((n, w), jnp.float32),
            pltpu.VMEM((n,), jnp.float32),
            pltpu.VMEM((n,), jnp.float32),
            pltpu.SemaphoreType.DMA,
        ],
        compiler_params=pltpu.CompilerParams(
            needs_layout_passes=False,
            skip_device_barrier=True,
        ),
    )
    return call(Y.reshape(n * h, w), ridx, x, xb)


def _copy_body(y_blk, o_blk):
    o_blk[...] = y_blk[...]


def _patch_body(x_ref, y_ref, v_ref, c_hbm, o_hbm, rowbuf, sem):
    n = rowbuf.shape[0]
    w = rowbuf.shape[-1]
    reads = []
    for i in range(n):
        d = pltpu.make_async_copy(
            o_hbm.at[i, pl.ds(y_ref[i], 1), :], rowbuf.at[i], sem
        )
        d.start()
        reads.append(d)
    cols = jax.lax.broadcasted_iota(jnp.int32, (1, w), 1)
    writes = []
    for i in range(n):
        reads[i].wait()
        rowbuf[i] = rowbuf[i] + jnp.where(cols == x_ref[i], v_ref[0], 0.0)
        d = pltpu.make_async_copy(
            rowbuf.at[i], o_hbm.at[i, pl.ds(y_ref[i], 1), :], sem
        )
        d.start()
        writes.append(d)
    for d in writes:
        d.wait()


def kernel(Y, X, x, y):
    n, h, w = Y.shape
    vals = X
    clone = pl.pallas_call(
        _copy_body,
        grid=(n, h // _BH),
        in_specs=[pl.BlockSpec((1, _BH, w), lambda i, j: (i, j, 0))],
        out_specs=pl.BlockSpec((1, _BH, w), lambda i, j: (i, j, 0)),
        out_shape=jax.ShapeDtypeStruct(Y.shape, Y.dtype),
        compiler_params=pltpu.CompilerParams(
            dimension_semantics=("parallel", "parallel"),
        ),
    )(Y)
    return pl.pallas_call(
        _patch_body,
        in_specs=[
            pl.BlockSpec(memory_space=pltpu.SMEM),
            pl.BlockSpec(memory_space=pltpu.SMEM),
            pl.BlockSpec(memory_space=pltpu.SMEM),
            pl.BlockSpec(memory_space=pl.ANY),
        ],
        out_specs=pl.BlockSpec(memory_space=pl.ANY),
        out_shape=jax.ShapeDtypeStruct(Y.shape, Y.dtype),
        input_output_aliases={3: 0},
        scratch_shapes=[
            pltpu.VMEM((n, 1, w), jnp.float32),
            pltpu.SemaphoreType.DMA,
        ],
    )(x, y, vals, clone)


# speedup vs baseline: 2.2374x; 1.0150x over previous
import jax
import jax.numpy as jnp
from jax.experimental import pallas as pl
from jax.experimental.pallas import tpu as pltpu

_BH = 1024


def _body(x_ref, y_ref, X_ref, y_blk, o_blk):
    i = pl.program_id(0)
    j = pl.program_id(1)
    o_blk[...] = y_blk[...]
    r_loc = y_ref[i] - j * _BH
    c = x_ref[i]

    @pl.when((r_loc >= 0) & (r_loc < _BH))
    def _inject():
        row = y_blk[0, pl.ds(r_loc, 1), :]
        w = row.shape[-1]
        colmask = jax.lax.broadcasted_iota(jnp.int32, (1, w), 1) == c
        o_blk[0, pl.ds(r_loc, 1), :] = row + jnp.where(colmask, X_ref[0], 0.0)


def kernel(Y, X, x, y):
    n, h, w = Y.shape
    grid = (n, h // _BH)
    return pl.pallas_call(
        _body,
        grid=grid,
        in_specs=[
            pl.BlockSpec(memory_space=pltpu.SMEM),
            pl.BlockSpec(memory_space=pltpu.SMEM),
            pl.BlockSpec(memory_space=pltpu.SMEM),
            pl.BlockSpec((1, _BH, w), lambda i, j: (i, j, 0)),
        ],
        out_specs=pl.BlockSpec((1, _BH, w), lambda i, j: (i, j, 0)),
        out_shape=jax.ShapeDtypeStruct(Y.shape, Y.dtype),
        compiler_params=pltpu.CompilerParams(
            dimension_semantics=("parallel", "parallel"),
        ),
    )(x, y, X, Y)


# speedup vs baseline: 2.2457x; 1.0037x over previous
import jax
import jax.numpy as jnp
from jax.experimental import pallas as pl
from jax.experimental.pallas import tpu as pltpu

_NB = 3


def _body(x_ref, y_ref, X_ref, y_hbm, o_hbm, bufs, rsem, wsem):
    n, h, w = y_hbm.shape

    def rd(k, b):
        return pltpu.make_async_copy(y_hbm.at[k], bufs.at[b], rsem.at[b])

    def wr(k, b):
        return pltpu.make_async_copy(bufs.at[b], o_hbm.at[k], wsem.at[b])

    for k in range(min(_NB, n)):
        rd(k, k).start()
    cols = jax.lax.broadcasted_iota(jnp.int32, (1, w), 1)
    for k in range(n):
        b = k % _NB
        rd(k, b).wait()
        r = y_ref[k]
        row = bufs[b, pl.ds(r, 1), :]
        bufs[b, pl.ds(r, 1), :] = row + jnp.where(
            cols == x_ref[k], X_ref[0], 0.0
        )
        wr(k, b).start()
        nk = k + _NB
        if nk < n:
            wr(k, b).wait()
            rd(nk, b).start()
    for k in range(max(n - _NB, 0), n):
        wr(k, k % _NB).wait()


def kernel(Y, X, x, y):
    n, h, w = Y.shape
    return pl.pallas_call(
        _body,
        in_specs=[
            pl.BlockSpec(memory_space=pltpu.SMEM),
            pl.BlockSpec(memory_space=pltpu.SMEM),
            pl.BlockSpec(memory_space=pltpu.SMEM),
            pl.BlockSpec(memory_space=pl.ANY),
        ],
        out_specs=pl.BlockSpec(memory_space=pl.ANY),
        out_shape=jax.ShapeDtypeStruct(Y.shape, Y.dtype),
        scratch_shapes=[
            pltpu.VMEM((_NB, h, w), jnp.float32),
            pltpu.SemaphoreType.DMA((_NB,)),
            pltpu.SemaphoreType.DMA((_NB,)),
        ],
        compiler_params=pltpu.CompilerParams(
            vmem_limit_bytes=56 * 1024 * 1024,
        ),
    )(x, y, X, Y)
